# baseline (speedup 1.0000x reference)
import functools

import jax
import jax.numpy as jnp
from jax import lax
from jax.experimental import pallas as pl
from jax.experimental.pallas import tpu as pltpu
from jax.experimental.pallas import tpu_sc as plsc

_VOCAB = 1000000
_DIM = 64
_NCLASS = 10
_SEQ = 50
_BATCH = 4096

_CPAD = 16
_GROUPS = 4
_PROWS = 262144
_GSHIFT = 2
_BK = 16384
_GRID = _PROWS // _BK

_NC = 2
_NS = 16
_NW = _NC * _NS
_BPW = _BATCH // _NW


def _proj_body(*refs):
    xs = refs[:_GROUPS]
    w_ref, o_ref = refs[_GROUPS], refs[_GROUPS + 1]
    xb = jnp.concatenate([x[...] for x in xs], axis=0)
    o_ref[...] = lax.dot_general(
        xb, w_ref[...], (((0,), (0,)), ((), ())),
        preferred_element_type=jnp.float32,
    )


_proj = pl.pallas_call(
    _proj_body,
    grid=(_GRID,),
    in_specs=[
        pl.BlockSpec(
            (_DIM, _BK),
            functools.partial(
                lambda i, s: (0, jnp.minimum(_GRID * s + i, _VOCAB // _BK)), s=s
            ),
        )
        for s in range(_GROUPS)
    ] + [pl.BlockSpec((_GROUPS * _DIM, _GROUPS * _CPAD), lambda i: (0, 0))],
    out_specs=pl.BlockSpec((_BK, _GROUPS * _CPAD), lambda i: (i, 0)),
    out_shape=jax.ShapeDtypeStruct((_PROWS, _GROUPS * _CPAD), jnp.float32),
    compiler_params=pltpu.CompilerParams(
        dimension_semantics=("arbitrary",),
        fuse_transposed_lhs_in_matmul=True,
    ),
)


_mesh = plsc.VectorSubcoreMesh(core_axis_name="c", subcore_axis_name="s")


@functools.partial(
    pl.kernel,
    mesh=_mesh,
    out_type=jax.ShapeDtypeStruct((_BATCH, _CPAD), jnp.float32),
    scratch_types=[
        pltpu.VMEM((_SEQ, _BPW), jnp.int32),
        pltpu.VMEM((_SEQ, _BPW), jnp.int32),
        pltpu.VMEM((_BPW, _CPAD), jnp.float32),
        pltpu.VMEM((_BPW, _CPAD), jnp.float32),
        pltpu.VMEM((_CPAD,), jnp.float32),
        pltpu.SemaphoreType.DMA,
        pltpu.SemaphoreType.DMA,
    ],
    compiler_params=pltpu.CompilerParams(use_tc_tiling_on_sc=False),
)
def _sc_embed_sum(idx_hbm, ptab_hbm, bias_hbm, out_hbm, idx_v, row_v, acc_v,
                  accb_v, bias_v, sem0, sem1):
    wid = lax.axis_index("s") * _NC + lax.axis_index("c")
    base = wid * _BPW
    pltpu.sync_copy(idx_hbm.at[:, pl.ds(base, _BPW)], idx_v)
    pltpu.sync_copy(bias_hbm, bias_v)

    def remap(l, _):
        for k in range(_BPW // 16):
            v = idx_v[l, pl.ds(16 * k, 16)]
            row_v[l, pl.ds(16 * k, 16)] = (
                (v & (_PROWS - 1)) << _GSHIFT) | lax.shift_right_logical(v, 18)
        return 0

    lax.fori_loop(0, _SEQ, remap, 0)

    c0 = pltpu.async_copy(ptab_hbm.at[row_v.at[0]], acc_v, sem0)
    c1 = pltpu.async_copy(ptab_hbm.at[row_v.at[1]], accb_v, sem1)
    c0.wait()
    c1.wait()
    copies = []
    for l in range(2, _SEQ):
        dst = acc_v if l % 2 == 0 else accb_v
        sem = sem0 if l % 2 == 0 else sem1
        copies.append(pltpu.async_copy(ptab_hbm.at[row_v.at[l]], dst, sem, add=True))
    for cp in copies:
        cp.wait()

    bvec = bias_v[...]
    def merge_bias(i, _):
        acc_v[i, :] = acc_v[i, :] + accb_v[i, :] + bvec
        return 0

    lax.fori_loop(0, _BPW, merge_bias, 0)
    pltpu.sync_copy(acc_v, out_hbm.at[pl.ds(base, _BPW)])


def kernel(input, table, W, b):
    idx = input[0].astype(jnp.int32)
    tableT = table.T
    ws = W * (1.0 / _SEQ)
    w8 = jnp.zeros((_GROUPS * _DIM, _GROUPS * _CPAD), jnp.float32)
    for s in range(_GROUPS):
        w8 = w8.at[s * _DIM:(s + 1) * _DIM, s * _CPAD:s * _CPAD + _NCLASS].set(ws)
    b16 = jnp.zeros((_CPAD,), jnp.float32).at[:_NCLASS].set(b)
    ptab = _proj(*([tableT] * _GROUPS), w8)
    ptab16 = ptab.reshape(_GROUPS * _PROWS, _CPAD)
    out = _sc_embed_sum(idx, ptab16, b16)
    return out[:, :_NCLASS]


# speedup vs baseline: 1.8272x; 1.8272x over previous
import functools

import jax
import jax.numpy as jnp
from jax import lax
from jax.experimental import pallas as pl
from jax.experimental.pallas import tpu as pltpu
from jax.experimental.pallas import tpu_sc as plsc

_VOCAB = 1000000
_DIM = 64
_NCLASS = 10
_SEQ = 50
_BATCH = 4096

_CPAD = 16
_GROUPS = 8
_PROWS = 131072
_BK = 8192
_GRID = _PROWS // _BK

_NC = 2
_NS = 16
_NW = _NC * _NS
_BPW = _BATCH // _NW


def _proj_body(*refs):
    xs = refs[:_GROUPS]
    w_ref, o_ref = refs[_GROUPS], refs[_GROUPS + 1]
    xb = jnp.concatenate([x[...] for x in xs], axis=0)
    o_ref[...] = lax.dot_general(
        xb, w_ref[...], (((0,), (0,)), ((), ())),
        preferred_element_type=jnp.float32,
    )


_proj = pl.pallas_call(
    _proj_body,
    grid=(_GRID,),
    in_specs=[
        pl.BlockSpec(
            (_DIM, _BK),
            functools.partial(
                lambda i, s: (0, jnp.minimum(_GRID * s + i, _VOCAB // _BK)), s=s
            ),
        )
        for s in range(_GROUPS)
    ] + [pl.BlockSpec((_GROUPS * _DIM, _GROUPS * _CPAD), lambda i: (0, 0))],
    out_specs=pl.BlockSpec((_BK, _GROUPS * _CPAD), lambda i: (i, 0)),
    out_shape=jax.ShapeDtypeStruct((_PROWS, _GROUPS * _CPAD), jnp.float32),
    compiler_params=pltpu.CompilerParams(
        dimension_semantics=("arbitrary",),
        fuse_transposed_lhs_in_matmul=True,
    ),
)


_mesh = plsc.VectorSubcoreMesh(core_axis_name="c", subcore_axis_name="s")


@functools.partial(
    pl.kernel,
    mesh=_mesh,
    out_type=jax.ShapeDtypeStruct((_BATCH, _CPAD), jnp.float32),
    scratch_types=[
        pltpu.VMEM((_SEQ, _BPW), jnp.int32),
        pltpu.VMEM((_SEQ, _BPW), jnp.int32),
        pltpu.VMEM((_BPW, _CPAD), jnp.float32),
        pltpu.VMEM((_BPW, _CPAD), jnp.float32),
        pltpu.VMEM((_CPAD,), jnp.float32),
        pltpu.SemaphoreType.DMA,
        pltpu.SemaphoreType.DMA,
    ],
    compiler_params=pltpu.CompilerParams(use_tc_tiling_on_sc=False),
)
def _sc_embed_sum(idx_hbm, ptab_hbm, bias_hbm, out_hbm, idx_v, row_v, acc_v,
                  accb_v, bias_v, sem0, sem1):
    wid = lax.axis_index("s") * _NC + lax.axis_index("c")
    base = wid * _BPW
    pltpu.sync_copy(idx_hbm.at[:, pl.ds(base, _BPW)], idx_v)
    pltpu.sync_copy(bias_hbm, bias_v)

    def remap(l, _):
        for k in range(_BPW // 16):
            v = idx_v[l, pl.ds(16 * k, 16)]
            row_v[l, pl.ds(16 * k, 16)] = (
                (v & (_PROWS - 1)) << 3) | lax.shift_right_logical(v, 17)
        return 0

    lax.fori_loop(0, _SEQ, remap, 0)

    c0 = pltpu.async_copy(ptab_hbm.at[row_v.at[0]], acc_v, sem0)
    c1 = pltpu.async_copy(ptab_hbm.at[row_v.at[1]], accb_v, sem1)
    c0.wait()
    c1.wait()
    copies = []
    for l in range(2, _SEQ):
        dst = acc_v if l % 2 == 0 else accb_v
        sem = sem0 if l % 2 == 0 else sem1
        copies.append(pltpu.async_copy(ptab_hbm.at[row_v.at[l]], dst, sem, add=True))
    for cp in copies:
        cp.wait()

    bvec = bias_v[...]
    def merge_bias(i, _):
        acc_v[i, :] = acc_v[i, :] + accb_v[i, :] + bvec
        return 0

    lax.fori_loop(0, _BPW, merge_bias, 0)
    pltpu.sync_copy(acc_v, out_hbm.at[pl.ds(base, _BPW)])


def kernel(input, table, W, b):
    idx = input[0].astype(jnp.int32)
    tableT = table.T
    ws = W * (1.0 / _SEQ)
    w8 = jnp.zeros((_GROUPS * _DIM, _GROUPS * _CPAD), jnp.float32)
    for s in range(_GROUPS):
        w8 = w8.at[s * _DIM:(s + 1) * _DIM, s * _CPAD:s * _CPAD + _NCLASS].set(ws)
    b16 = jnp.zeros((_CPAD,), jnp.float32).at[:_NCLASS].set(b)
    ptab = _proj(*([tableT] * _GROUPS), w8)
    ptab16 = ptab.reshape(_GROUPS * _PROWS, _CPAD)
    out = _sc_embed_sum(idx, ptab16, b16)
    return out[:, :_NCLASS]
